# Initial kernel scaffold; baseline (speedup 1.0000x reference)
import jax
import jax.numpy as jnp
from jax.experimental import pallas as pl


def kernel(x, Wq, bq, Wk, bk, Wv, bv, conv_w, conv_b, pw_w, pw_b):
    raise NotImplementedError("write your pallas kernel here")


# speedup vs baseline: 5.6130x; 5.6130x over previous
import functools
import math

import jax
import jax.numpy as jnp
from jax.experimental import pallas as pl

_IN_CH = 32
_OUT_CH = 32
_K = 8
_SCALE = 4
_H = 128
_W = 128
_C1 = (_IN_CH + 2) * _SCALE * _SCALE
_N = (_H * _W) // (_SCALE * _SCALE)
_P = _SCALE * _SCALE
_CF = _OUT_CH * _P


def _fold_kernel(conv_w_t_ref, cb2_ref, pw_w_ref, pw_b_ref, wfin_ref, bfin_ref):
    pw = pw_w_ref[...]
    ii = jax.lax.broadcasted_iota(jnp.int32, (_C1, _CF), 0)
    qq = jax.lax.broadcasted_iota(jnp.int32, (_C1, _CF), 1)
    same_p = (ii % _P) == (qq % _P)
    oc_c = (jax.lax.broadcasted_iota(jnp.int32, (_C1, _IN_CH + 2), 0) // _P ==
            jax.lax.broadcasted_iota(jnp.int32, (_C1, _IN_CH + 2), 1)
            ).astype(jnp.float32)
    oc_o = (jax.lax.broadcasted_iota(jnp.int32, (_OUT_CH, _CF), 1) // _P ==
            jax.lax.broadcasted_iota(jnp.int32, (_OUT_CH, _CF), 0)
            ).astype(jnp.float32)
    pw_big = jnp.dot(jnp.dot(oc_c, pw.T, preferred_element_type=jnp.float32),
                     oc_o, preferred_element_type=jnp.float32)
    pw_big = jnp.where(same_p, pw_big, 0.0)
    for k in range(_K):
        wfin_ref[k] = jnp.dot(conv_w_t_ref[k], pw_big,
                              preferred_element_type=jnp.float32)
    bf2 = jnp.dot(pw, cb2_ref[...], preferred_element_type=jnp.float32)
    bfin_ref[...] = bf2 + pw_b_ref[...].reshape(_OUT_CH, 1)


def _attn_kernel(x_ref, wq_ref, bq_ref, wk_ref, bk_ref, wv_ref, bv_ref,
                 wfin_ref, bfin_ref, out_ref):
    x = x_ref[0]
    q = jnp.dot(x, wq_ref[...], preferred_element_type=jnp.float32) + bq_ref[...]
    k = jnp.dot(x, wk_ref[...], preferred_element_type=jnp.float32) + bk_ref[...]
    v = jnp.dot(x, wv_ref[...], preferred_element_type=jnp.float32) + bv_ref[...]
    sim = jax.lax.dot_general(q, k, (((1,), (1,)), ((), ())),
                              preferred_element_type=jnp.float32)
    sim = sim * (1.0 / math.sqrt(_C1))

    iota_m = jax.lax.broadcasted_iota(jnp.int32, (_N, _N), 1)
    topv = []
    topi = []
    work = sim
    for _ in range(_K):
        mx = jnp.max(work, axis=1, keepdims=True)
        idx = jnp.min(jnp.where(work == mx, iota_m, _N), axis=1,
                      keepdims=True)
        topv.append(mx)
        topi.append(idx)
        work = jnp.where(iota_m == idx, -jnp.inf, work)

    exps = [jnp.exp(tv - topv[0]) for tv in topv]
    denom = functools.reduce(lambda a, b: a + b, exps)
    inv = 1.0 / denom

    acc = jnp.broadcast_to(bfin_ref[...], (_N, _CF))
    for kk in range(_K):
        u_k = jnp.dot(v, wfin_ref[kk], preferred_element_type=jnp.float32)
        a_k = exps[kk] * inv
        p_k = jnp.where(iota_m == topi[kk], a_k, 0.0)
        acc = acc + jnp.dot(p_k, u_k, preferred_element_type=jnp.float32)
    out_ref[0] = acc


def kernel(x, Wq, bq, Wk, bk, Wv, bv, conv_w, conv_b, pw_w, pw_b):
    b = x.shape[0]
    xg, yg = jnp.meshgrid(jnp.arange(_H, dtype=jnp.float32),
                          jnp.arange(_W, dtype=jnp.float32), indexing='ij')
    xy = jnp.stack([xg, yg], axis=0)
    norm = jnp.sqrt(jnp.sum(xy * xy, axis=0, keepdims=True))
    xy = xy / jnp.maximum(norm, 1e-12)
    coords = jnp.broadcast_to(xy[None], (b, 2, _H, _W))
    xc = jnp.concatenate([x, coords], axis=1)
    x1 = xc.reshape(b, _IN_CH + 2, _H // _SCALE, _SCALE, _W // _SCALE, _SCALE)
    x1 = x1.transpose(0, 1, 3, 5, 2, 4).reshape(b, _C1, _N)
    x2t = x1.transpose(0, 2, 1)

    conv_w_t = conv_w.transpose(2, 1, 0)
    cb2 = conv_b.reshape(_IN_CH + 2, _P)
    wfin, bfin2 = pl.pallas_call(
        _fold_kernel,
        out_shape=(
            jax.ShapeDtypeStruct((_K, _C1, _CF), jnp.float32),
            jax.ShapeDtypeStruct((_OUT_CH, _P), jnp.float32),
        ),
    )(conv_w_t, cb2, pw_w, pw_b.reshape(_OUT_CH, 1))
    bfin = bfin2.reshape(1, _CF)

    final = pl.pallas_call(
        _attn_kernel,
        grid=(b,),
        in_specs=[
            pl.BlockSpec((1, _N, _C1), lambda i: (i, 0, 0)),
            pl.BlockSpec((_C1, _C1), lambda i: (0, 0)),
            pl.BlockSpec((1, _C1), lambda i: (0, 0)),
            pl.BlockSpec((_C1, _C1), lambda i: (0, 0)),
            pl.BlockSpec((1, _C1), lambda i: (0, 0)),
            pl.BlockSpec((_C1, _C1), lambda i: (0, 0)),
            pl.BlockSpec((1, _C1), lambda i: (0, 0)),
            pl.BlockSpec((_K, _C1, _CF), lambda i: (0, 0, 0)),
            pl.BlockSpec((1, _CF), lambda i: (0, 0)),
        ],
        out_specs=pl.BlockSpec((1, _N, _CF), lambda i: (i, 0, 0)),
        out_shape=jax.ShapeDtypeStruct((b, _N, _CF), jnp.float32),
    )(x2t, Wq.T, bq.reshape(1, _C1), Wk.T, bk.reshape(1, _C1),
      Wv.T, bv.reshape(1, _C1), wfin, bfin)

    out = final.reshape(b, _H // _SCALE, _W // _SCALE, _OUT_CH, _SCALE, _SCALE)
    out = out.transpose(0, 3, 1, 4, 2, 5).reshape(b, _OUT_CH, _H, _W)
    return out


# speedup vs baseline: 6.0826x; 1.0837x over previous
import functools
import math

import jax
import jax.numpy as jnp
from jax.experimental import pallas as pl

_IN_CH = 32
_OUT_CH = 32
_K = 8
_SCALE = 4
_H = 128
_W = 128
_C1 = (_IN_CH + 2) * _SCALE * _SCALE
_N = (_H * _W) // (_SCALE * _SCALE)
_P = _SCALE * _SCALE
_CF = _OUT_CH * _P


def _fold_kernel(conv_w_t_ref, cb2_ref, pw_w_ref, pw_b_ref, wfin_ref, bfin_ref):
    pw = pw_w_ref[...]
    ii = jax.lax.broadcasted_iota(jnp.int32, (_C1, _CF), 0)
    qq = jax.lax.broadcasted_iota(jnp.int32, (_C1, _CF), 1)
    same_p = (ii % _P) == (qq % _P)
    oc_c = (jax.lax.broadcasted_iota(jnp.int32, (_C1, _IN_CH + 2), 0) // _P ==
            jax.lax.broadcasted_iota(jnp.int32, (_C1, _IN_CH + 2), 1)
            ).astype(jnp.float32)
    oc_o = (jax.lax.broadcasted_iota(jnp.int32, (_OUT_CH, _CF), 1) // _P ==
            jax.lax.broadcasted_iota(jnp.int32, (_OUT_CH, _CF), 0)
            ).astype(jnp.float32)
    pw_big = jnp.dot(jnp.dot(oc_c, pw.T, preferred_element_type=jnp.float32),
                     oc_o, preferred_element_type=jnp.float32)
    pw_big = jnp.where(same_p, pw_big, 0.0)
    for k in range(_K):
        wfin_ref[k] = jnp.dot(conv_w_t_ref[k], pw_big,
                              preferred_element_type=jnp.float32)
    bf2 = jnp.dot(pw, cb2_ref[...], preferred_element_type=jnp.float32)
    bfin_ref[...] = bf2 + pw_b_ref[...].reshape(_OUT_CH, 1)


def _attn_kernel(x_ref, wq_ref, bq_ref, wk_ref, bk_ref, wv_ref, bv_ref,
                 wfin_ref, bfin_ref, out_ref):
    x = x_ref[0]
    q = jnp.dot(x, wq_ref[...], preferred_element_type=jnp.float32) + bq_ref[...]
    k = jnp.dot(x, wk_ref[...], preferred_element_type=jnp.float32) + bk_ref[...]
    v = jnp.dot(x, wv_ref[...], preferred_element_type=jnp.float32) + bv_ref[...]
    sim = jax.lax.dot_general(q, k, (((1,), (1,)), ((), ())),
                              preferred_element_type=jnp.float32)
    sim = sim * (1.0 / math.sqrt(_C1))

    iota_m = jax.lax.broadcasted_iota(jnp.int32, (_N, _N), 1)
    topv = []
    topi = []
    work = sim
    for _ in range(_K):
        mx = jnp.max(work, axis=1, keepdims=True)
        idx = jnp.min(jnp.where(work == mx, iota_m, _N), axis=1,
                      keepdims=True)
        topv.append(mx)
        topi.append(idx)
        work = jnp.where(iota_m == idx, -jnp.inf, work)

    exps = [jnp.exp(tv - topv[0]) for tv in topv]
    denom = functools.reduce(lambda a, b: a + b, exps)
    inv = 1.0 / denom

    acc = jnp.broadcast_to(bfin_ref[...], (_N, _CF))
    for kk in range(_K):
        u_k = jnp.dot(v, wfin_ref[kk], preferred_element_type=jnp.float32)
        a_k = exps[kk] * inv
        p_k = jnp.where(iota_m == topi[kk], 1.0, 0.0).astype(jnp.bfloat16)
        g_k = jnp.dot(p_k, u_k.astype(jnp.bfloat16),
                      preferred_element_type=jnp.float32)
        acc = acc + g_k * a_k
    out_ref[0] = acc


def kernel(x, Wq, bq, Wk, bk, Wv, bv, conv_w, conv_b, pw_w, pw_b):
    b = x.shape[0]
    xg, yg = jnp.meshgrid(jnp.arange(_H, dtype=jnp.float32),
                          jnp.arange(_W, dtype=jnp.float32), indexing='ij')
    xy = jnp.stack([xg, yg], axis=0)
    norm = jnp.sqrt(jnp.sum(xy * xy, axis=0, keepdims=True))
    xy = xy / jnp.maximum(norm, 1e-12)
    coords = jnp.broadcast_to(xy[None], (b, 2, _H, _W))
    xc = jnp.concatenate([x, coords], axis=1)
    x1 = xc.reshape(b, _IN_CH + 2, _H // _SCALE, _SCALE, _W // _SCALE, _SCALE)
    x1 = x1.transpose(0, 1, 3, 5, 2, 4).reshape(b, _C1, _N)
    x2t = x1.transpose(0, 2, 1)

    conv_w_t = conv_w.transpose(2, 1, 0)
    cb2 = conv_b.reshape(_IN_CH + 2, _P)
    wfin, bfin2 = pl.pallas_call(
        _fold_kernel,
        out_shape=(
            jax.ShapeDtypeStruct((_K, _C1, _CF), jnp.float32),
            jax.ShapeDtypeStruct((_OUT_CH, _P), jnp.float32),
        ),
    )(conv_w_t, cb2, pw_w, pw_b.reshape(_OUT_CH, 1))
    bfin = bfin2.reshape(1, _CF)

    final = pl.pallas_call(
        _attn_kernel,
        grid=(b,),
        in_specs=[
            pl.BlockSpec((1, _N, _C1), lambda i: (i, 0, 0)),
            pl.BlockSpec((_C1, _C1), lambda i: (0, 0)),
            pl.BlockSpec((1, _C1), lambda i: (0, 0)),
            pl.BlockSpec((_C1, _C1), lambda i: (0, 0)),
            pl.BlockSpec((1, _C1), lambda i: (0, 0)),
            pl.BlockSpec((_C1, _C1), lambda i: (0, 0)),
            pl.BlockSpec((1, _C1), lambda i: (0, 0)),
            pl.BlockSpec((_K, _C1, _CF), lambda i: (0, 0, 0)),
            pl.BlockSpec((1, _CF), lambda i: (0, 0)),
        ],
        out_specs=pl.BlockSpec((1, _N, _CF), lambda i: (i, 0, 0)),
        out_shape=jax.ShapeDtypeStruct((b, _N, _CF), jnp.float32),
    )(x2t, Wq.T, bq.reshape(1, _C1), Wk.T, bk.reshape(1, _C1),
      Wv.T, bv.reshape(1, _C1), wfin, bfin)

    out = final.reshape(b, _H // _SCALE, _W // _SCALE, _OUT_CH, _SCALE, _SCALE)
    out = out.transpose(0, 3, 1, 4, 2, 5).reshape(b, _OUT_CH, _H, _W)
    return out
